# Initial kernel scaffold; baseline (speedup 1.0000x reference)
import jax
import jax.numpy as jnp
from jax.experimental import pallas as pl


def kernel(v_feat, edge_index, preference, W_mlp, b_mlp, W_conv1, W_lin1, b_lin1, W_g1, b_g1, W_conv2, W_lin2, b_lin2, W_g2, b_g2, W_conv3, W_lin3, b_lin3, W_g3, b_g3, id_embedding):
    raise NotImplementedError("write your pallas kernel here")


# speedup vs baseline: 7.7833x; 7.7833x over previous
import functools

import jax
import jax.numpy as jnp
from jax import lax
from jax.experimental import pallas as pl
from jax.experimental.pallas import tpu as pltpu
from jax.experimental.pallas import tpu_sc as plsc

NUM_USER = 2000
NUM_ITEM = 8000
N = NUM_USER + NUM_ITEM
E_DIR = 2 * 320000
NC, NS, L = 2, 16, 16
CHUNK = 128
N_PAD = 10112
ZROWS = N_PAD // NS
ROWS_PT = 624

KA = (E_DIR + NS * CHUNK - 1) // (NS * CHUNK)
PAD_A = NS * KA * CHUNK - E_DIR
KB = (E_DIR + NC * NS * CHUNK - 1) // (NC * NS * CHUNK)
PAD_B = NC * NS * KB * CHUNK - E_DIR

_lr = functools.partial(jax.nn.leaky_relu, negative_slope=0.01)


def _tc_a_body(pref, vfeat, wmlp, bmlp, wconv1, wlin1, blin1, idemb,
               xw_st, xhat):
    i = pl.program_id(0)
    tf = jnp.dot(vfeat[...], wmlp[...].T, preferred_element_type=jnp.float32)
    tf = tf + bmlp[...]
    x = jnp.where(i < 2, pref[...], tf)
    nrm = jnp.sqrt(jnp.sum(x * x, axis=1, keepdims=True))
    x = x / jnp.maximum(nrm, 1e-12)
    xw = jnp.dot(x, wconv1[...], preferred_element_type=jnp.float32)
    xw_st[...] = jnp.stack(
        [xw[:, 64 * k:64 * (k + 1)] for k in range(4)], axis=0)
    xh = _lr(jnp.dot(x, wlin1[...].T, preferred_element_type=jnp.float32)
             + blin1[...])
    xhat[...] = xh + idemb[...]


def _tc_b_body(h01, h23, xhat_in, wg, bg, wconv, wlin, blin, idemb,
               xw_out, xhat_out):
    h = jnp.concatenate([h01[0], h01[1], h23[0], h23[1]], axis=1)
    h = _lr(h)
    _tc_mid_tail(h, xhat_in, wg, bg, wconv, wlin, blin, idemb,
                 xw_out, xhat_out)


def _tc_c_body(h_in, xhat_in, wg, bg, wconv, wlin, blin, idemb,
               xw_out, xhat_out):
    h = _lr(h_in[0] + h_in[1])
    _tc_mid_tail(h, xhat_in, wg, bg, wconv, wlin, blin, idemb,
                 xw_out, xhat_out)


def _tc_mid_tail(h, xhat_in, wg, bg, wconv, wlin, blin, idemb,
                 xw_out, xhat_out):
    g = jnp.dot(h, wg[...].T, preferred_element_type=jnp.float32) + bg[...]
    x = _lr(g + xhat_in[...])
    xw_out[...] = jnp.dot(x, wconv[...], preferred_element_type=jnp.float32)
    xh = _lr(jnp.dot(x, wlin[...].T, preferred_element_type=jnp.float32)
             + blin[...])
    xhat_out[...] = xh + idemb[...]


def _tc_d_body(h_in, xhat_in, wg, bg, out):
    h = _lr(h_in[0] + h_in[1])
    g = jnp.dot(h, wg[...].T, preferred_element_type=jnp.float32) + bg[...]
    out[...] = _lr(g + xhat_in[...])


_BLK = 1000
_GRID = N // _BLK


def _full(shape):
    return pl.BlockSpec(shape, lambda i: tuple(0 for _ in shape))


def _rows(width):
    return pl.BlockSpec((_BLK, width), lambda i: (i, 0))


def _stacked(width):
    return pl.BlockSpec((2, _BLK, width), lambda i: (0, i, 0))


def _sc_conv_body(n_chunks, xw, src, dst, zeros, out,
                  idx_src, idx_dst, rows, accum, sem):
    c = lax.axis_index("c")
    s = lax.axis_index("s")
    w = c * NS + s
    pltpu.sync_copy(src.at[w], idx_src)
    pltpu.sync_copy(dst.at[w], idx_dst)
    pltpu.sync_copy(zeros, accum.at[pl.ds(s * ZROWS, ZROWS)])
    plsc.subcore_barrier()

    def body(j, carry):
        pltpu.async_copy(xw.at[idx_src.at[j]], rows, sem).wait()
        pltpu.sync_copy(rows, accum.at[idx_dst.at[j]], add=True)
        return carry

    lax.fori_loop(0, n_chunks, body, 0)
    plsc.subcore_barrier()

    @pl.when(s < NS - 1)
    def _():
        pltpu.sync_copy(accum.at[pl.ds(s * ROWS_PT, ROWS_PT)],
                        out.at[pl.ds(c * N + s * ROWS_PT, ROWS_PT)])

    @pl.when(s == NS - 1)
    def _():
        last = (NS - 1) * ROWS_PT
        pltpu.sync_copy(accum.at[pl.ds(last, N - last)],
                        out.at[pl.ds(c * N + last, N - last)])


def _make_sc_conv(n_chunks):
    mesh = plsc.VectorSubcoreMesh(core_axis_name="c", subcore_axis_name="s")
    return pl.kernel(
        functools.partial(_sc_conv_body, n_chunks),
        out_type=jax.ShapeDtypeStruct((2 * N, 64), jnp.float32),
        mesh=mesh,
        compiler_params=pltpu.CompilerParams(use_tc_tiling_on_sc=False),
        scratch_types=[
            pltpu.VMEM((n_chunks, CHUNK), jnp.int32),
            pltpu.VMEM((n_chunks, CHUNK), jnp.int32),
            pltpu.VMEM((CHUNK, 64), jnp.float32),
            pltpu.VMEM_SHARED((N_PAD, 64), jnp.float32),
            pltpu.SemaphoreType.DMA,
        ],
    )


def kernel(v_feat, edge_index, preference, W_mlp, b_mlp,
           W_conv1, W_lin1, b_lin1, W_g1, b_g1,
           W_conv2, W_lin2, b_lin2, W_g2, b_g2,
           W_conv3, W_lin3, b_lin3, W_g3, b_g3,
           id_embedding):
    f32 = jnp.float32

    e0 = edge_index[:, 0]
    e1 = edge_index[:, 1]
    src_dir = jnp.concatenate([e0, e1])
    dst_dir = jnp.concatenate([e1, e0])

    src_a_t = jnp.concatenate(
        [src_dir, jnp.zeros((PAD_A,), jnp.int32)]).reshape(NS, KA, CHUNK)
    dst_a_t = jnp.concatenate(
        [dst_dir, jnp.full((PAD_A,), N, jnp.int32)]).reshape(NS, KA, CHUNK)
    src_a0 = jnp.concatenate([src_a_t, src_a_t + N], axis=0)
    src_a1 = jnp.concatenate([src_a_t + 2 * N, src_a_t + 3 * N], axis=0)
    dst_a = jnp.concatenate([dst_a_t, dst_a_t], axis=0)

    src_b = jnp.concatenate([src_dir, jnp.zeros((PAD_B,), jnp.int32)])
    dst_b = jnp.concatenate([dst_dir, jnp.full((PAD_B,), N, jnp.int32)])
    src_b = src_b.reshape(NC * NS, KB, CHUNK)
    dst_b = dst_b.reshape(NC * NS, KB, CHUNK)

    zeros_b = jnp.zeros((ZROWS, 64), f32)

    b_mlp2 = b_mlp.reshape(1, 256)
    b_lin1_2 = b_lin1.reshape(1, 64)
    b_g1_2 = b_g1.reshape(1, 64)
    b_lin2_2 = b_lin2.reshape(1, 64)
    b_g2_2 = b_g2.reshape(1, 64)
    b_lin3_2 = b_lin3.reshape(1, 64)
    b_g3_2 = b_g3.reshape(1, 64)

    xw_st, xhat1 = pl.pallas_call(
        _tc_a_body,
        grid=(_GRID,),
        in_specs=[
            pl.BlockSpec((_BLK, 256), lambda i: (jnp.minimum(i, 1), 0)),
            pl.BlockSpec((_BLK, 128), lambda i: (jnp.maximum(i - 2, 0), 0)),
            _full((256, 128)), _full((1, 256)), _full((256, 256)),
            _full((64, 256)), _full((1, 64)), _rows(64),
        ],
        out_specs=[pl.BlockSpec((4, _BLK, 64), lambda i: (0, i, 0)),
                   _rows(64)],
        out_shape=[jax.ShapeDtypeStruct((4, N, 64), f32),
                   jax.ShapeDtypeStruct((N, 64), f32)],
    )(preference, v_feat, W_mlp, b_mlp2, W_conv1, W_lin1, b_lin1_2,
      id_embedding)

    xw4 = xw_st.reshape(4 * N, 64)
    conv1 = _make_sc_conv(KA)
    h01 = conv1(xw4, src_a0, dst_a, zeros_b).reshape(2, N, 64)
    h23 = conv1(xw4, src_a1, dst_a, zeros_b).reshape(2, N, 64)

    xw2, xhat2 = pl.pallas_call(
        _tc_b_body,
        grid=(_GRID,),
        in_specs=[
            _stacked(64), _stacked(64), _rows(64),
            _full((64, 256)), _full((1, 64)), _full((64, 64)),
            _full((64, 64)), _full((1, 64)), _rows(64),
        ],
        out_specs=[_rows(64), _rows(64)],
        out_shape=[jax.ShapeDtypeStruct((N, 64), f32),
                   jax.ShapeDtypeStruct((N, 64), f32)],
    )(h01, h23, xhat1, W_g1, b_g1_2, W_conv2, W_lin2, b_lin2_2, id_embedding)

    h2_st = _make_sc_conv(KB)(xw2, src_b, dst_b, zeros_b).reshape(2, N, 64)

    xw3, xhat3 = pl.pallas_call(
        _tc_c_body,
        grid=(_GRID,),
        in_specs=[
            _stacked(64), _rows(64),
            _full((64, 64)), _full((1, 64)), _full((64, 64)),
            _full((64, 64)), _full((1, 64)), _rows(64),
        ],
        out_specs=[_rows(64), _rows(64)],
        out_shape=[jax.ShapeDtypeStruct((N, 64), f32),
                   jax.ShapeDtypeStruct((N, 64), f32)],
    )(h2_st, xhat2, W_g2, b_g2_2, W_conv3, W_lin3, b_lin3_2, id_embedding)

    h3_st = _make_sc_conv(KB)(xw3, src_b, dst_b, zeros_b).reshape(2, N, 64)

    out = pl.pallas_call(
        _tc_d_body,
        grid=(_GRID,),
        in_specs=[_stacked(64), _rows(64), _full((64, 64)), _full((1, 64))],
        out_specs=_rows(64),
        out_shape=jax.ShapeDtypeStruct((N, 64), f32),
    )(h3_st, xhat3, W_g3, b_g3_2)
    return out
